# baseline (speedup 1.0000x reference)
import functools

import jax
import jax.numpy as jnp
from jax import lax
from jax.experimental import pallas as pl
from jax.experimental.pallas import tpu as pltpu, tpu_sc as plsc

N = 10000
D = 128
E = 320000
THRESHOLD = 0.1

DS = D + 16
N_PAD = 10240
BLK = 512
GRID = N_PAD // BLK

NC = 2
NS = 16
NW = NC * NS
KB = 32
NBATCH = 324
EPW = KB * NBATCH
E_TOT = E + N
E_PAD = NW * EPW
ROWS_PER_TILE = N_PAD // NS

f32 = jnp.float32
i32 = jnp.int32


def _aux_cols(rn, rows):
    return jnp.concatenate(
        [rn, jnp.ones((rows, 1), f32), jnp.zeros((rows, 14), f32)], axis=1)


def _pre_body(x_ref, w_ref, xhat_ref, b_ref):
    x = x_ref[...]
    s = jnp.sum(x * x, axis=-1, keepdims=True)
    ni = jnp.maximum(jnp.sqrt(s), 1e-8)
    xhat_ref[...] = x / ni
    h = jnp.dot(x, w_ref[...], preferred_element_type=f32)
    rn = 1.0 / jnp.maximum(jnp.sqrt(jnp.sum(h * h, axis=-1, keepdims=True)), 1e-8)
    b_ref[:, 0:D] = h
    b_ref[:, D:DS] = _aux_cols(rn, BLK)


def _tc_pre(x_pad, W):
    return pl.pallas_call(
        _pre_body,
        grid=(GRID,),
        in_specs=[
            pl.BlockSpec((BLK, D), lambda i: (i, 0)),
            pl.BlockSpec((D, D), lambda i: (0, 0)),
        ],
        out_specs=[
            pl.BlockSpec((BLK, D), lambda i: (i, 0)),
            pl.BlockSpec((BLK, DS), lambda i: (i, 0)),
        ],
        out_shape=[
            jax.ShapeDtypeStruct((N_PAD, D), f32),
            jax.ShapeDtypeStruct((N_PAD, DS), f32),
        ],
    )(x_pad, W)


def _combine(acc_ref, b_ref):
    nm = acc_ref[0, :, 0:D] + acc_ref[1, :, 0:D]
    dd = acc_ref[0, :, D + 1:D + 2] + acc_ref[1, :, D + 1:D + 2] + 1e-16
    return nm / dd + b_ref[...]


def _mid_body(acc_ref, b_ref, lnw_ref, lnb_ref, w1_ref, xhat_ref, b2_ref):
    out = _combine(acc_ref, b_ref)
    mu = jnp.mean(out, axis=-1, keepdims=True)
    var = jnp.mean((out - mu) ** 2, axis=-1, keepdims=True)
    h1 = (out - mu) / jnp.sqrt(var + 1e-5) * lnw_ref[...] + lnb_ref[...]
    h1 = jnp.maximum(h1, 0.0)
    s = jnp.sum(h1 * h1, axis=-1, keepdims=True)
    ni = jnp.maximum(jnp.sqrt(s), 1e-8)
    xhat_ref[...] = h1 / ni
    h2 = jnp.dot(h1, w1_ref[...], preferred_element_type=f32)
    rn = 1.0 / jnp.maximum(jnp.sqrt(jnp.sum(h2 * h2, axis=-1, keepdims=True)), 1e-8)
    b2_ref[:, 0:D] = h2
    b2_ref[:, D:DS] = _aux_cols(rn, BLK)


def _tc_mid(acc, b0, ln_w, ln_b, W1):
    return pl.pallas_call(
        _mid_body,
        grid=(GRID,),
        in_specs=[
            pl.BlockSpec((NC, BLK, DS), lambda i: (0, i, 0)),
            pl.BlockSpec((1, D), lambda i: (0, 0)),
            pl.BlockSpec((1, D), lambda i: (0, 0)),
            pl.BlockSpec((1, D), lambda i: (0, 0)),
            pl.BlockSpec((D, D), lambda i: (0, 0)),
        ],
        out_specs=[
            pl.BlockSpec((BLK, D), lambda i: (i, 0)),
            pl.BlockSpec((BLK, DS), lambda i: (i, 0)),
        ],
        out_shape=[
            jax.ShapeDtypeStruct((N_PAD, D), f32),
            jax.ShapeDtypeStruct((N_PAD, DS), f32),
        ],
    )(acc, b0.reshape(1, D), ln_w.reshape(1, D), ln_b.reshape(1, D), W1)


def _final_body(acc_ref, b_ref, out_ref):
    out = _combine(acc_ref, b_ref)
    m = jnp.max(out, axis=-1, keepdims=True)
    z = out - m
    out_ref[...] = z - jnp.log(jnp.sum(jnp.exp(z), axis=-1, keepdims=True))


def _tc_final(acc, b1):
    return pl.pallas_call(
        _final_body,
        grid=(GRID,),
        in_specs=[
            pl.BlockSpec((NC, BLK, DS), lambda i: (0, i, 0)),
            pl.BlockSpec((1, D), lambda i: (0, 0)),
        ],
        out_specs=pl.BlockSpec((BLK, D), lambda i: (i, 0)),
        out_shape=jax.ShapeDtypeStruct((N_PAD, D), f32),
    )(acc, b1.reshape(1, D))


def _sc_body(xhat_hbm, srct_hbm, idx_hbm, zz_hbm,
             acc_hbm,
             idxb, bs0, bs1, bs2, bs3, xd0, xd1, xd2, xd3, spacc,
             gsem0, gsem1, gsem2, gsem3, ssem0, ssem1, ssem2, ssem3,
             isem0, isem1, isem2, isem3):
    c = lax.axis_index("c")
    s = lax.axis_index("s")
    wid = c * NS + s
    rbase = s * ROWS_PER_TILE

    bs = (bs0, bs1, bs2, bs3)
    xd = (xd0, xd1, xd2, xd3)
    gsem = (gsem0, gsem1, gsem2, gsem3)
    ssem = (ssem0, ssem1, ssem2, ssem3)
    isem = (isem0, isem1, isem2, isem3)

    pltpu.sync_copy(zz_hbm, spacc.at[pl.ds(rbase, ROWS_PER_TILE)])

    plsc.subcore_barrier()

    def compute(p):
        def edge_body(e, carry2):
            ch = [bs[p][e, pl.ds(cc * 16, 16)] for cc in range(8)]
            aux = bs[p][e, pl.ds(D, 16)]
            acc = ch[0] * xd[p][e, pl.ds(0, 16)]
            for cc in range(1, 8):
                acc = acc + ch[cc] * xd[p][e, pl.ds(cc * 16, 16)]
            w = jnp.sum(acc) * aux[0]
            w = jnp.where(w < THRESHOLD, f32(1e-6), w)
            for cc in range(8):
                bs[p][e, pl.ds(cc * 16, 16)] = ch[cc] * w
            bs[p][e, pl.ds(D, 16)] = aux * w
            return carry2

        lax.fori_loop(0, KB, edge_body, 0, unroll=4)

    def issue_gathers(b, p):
        sl = lax.rem(b, 8)
        pltpu.async_copy(srct_hbm.at[idxb.at[sl, 0]], bs[p], gsem[p])
        pltpu.async_copy(xhat_hbm.at[idxb.at[sl, 1]], xd[p], gsem[p])

    def drain_gathers(b, p):
        sl = lax.rem(b, 8)
        pltpu.make_async_copy(srct_hbm.at[idxb.at[sl, 0]], bs[p], gsem[p]).wait()
        pltpu.make_async_copy(xhat_hbm.at[idxb.at[sl, 1]], xd[p], gsem[p]).wait()

    def issue_scatter(b, p):
        sl = lax.rem(b, 8)
        pltpu.async_copy(bs[p], spacc.at[idxb.at[sl, 1]], ssem[p], add=True)

    def drain_scatter(b, p):
        sl = lax.rem(b, 8)
        pltpu.make_async_copy(bs[p], spacc.at[idxb.at[sl, 1]], ssem[p]).wait()

    def issue_idx(b, parity):
        sl = lax.rem(b, 8)
        pltpu.async_copy(idx_hbm.at[wid, b], idxb.at[sl], isem[parity])

    def drain_idx(b, parity):
        sl = lax.rem(b, 8)
        pltpu.make_async_copy(idx_hbm.at[wid, b], idxb.at[sl], isem[parity]).wait()

    pltpu.sync_copy(idx_hbm.at[wid, 0], idxb.at[0])
    pltpu.sync_copy(idx_hbm.at[wid, 1], idxb.at[1])
    issue_idx(2, 2)
    issue_idx(3, 3)
    issue_gathers(0, 0)
    issue_gathers(1, 1)

    def k_body(k, carry):
        for p in (0, 1, 2, 3):
            b = 4 * k + p

            def _prefetch_gathers():
                drain_idx(b + 2, (p + 2) % 4)
                issue_gathers(b + 2, (p + 2) % 4)

            drain_gathers(b, p)
            pl.when(b >= 2)(lambda: drain_scatter(b - 2, (p + 2) % 4))
            pl.when(b + 2 < NBATCH)(_prefetch_gathers)
            issue_idx(b + 4, p)
            compute(p)
            issue_scatter(b, p)
        return carry

    lax.fori_loop(0, NBATCH // 4, k_body, 0)

    drain_scatter(NBATCH - 2, 2)
    drain_scatter(NBATCH - 1, 3)
    for q in (0, 1, 2, 3):
        drain_idx(NBATCH + q, q)

    plsc.subcore_barrier()

    pltpu.sync_copy(spacc.at[pl.ds(rbase, ROWS_PER_TILE)],
                    acc_hbm.at[c, pl.ds(rbase, ROWS_PER_TILE)])


_sc_attn = functools.partial(
    pl.kernel,
    out_type=jax.ShapeDtypeStruct((NC, N_PAD, DS), f32),
    mesh=plsc.VectorSubcoreMesh(core_axis_name="c", subcore_axis_name="s"),
    compiler_params=pltpu.CompilerParams(needs_layout_passes=False,
                                         use_tc_tiling_on_sc=False),
    scratch_types=(
        [pltpu.VMEM((8, 2, KB), i32)]
        + [pltpu.VMEM((KB, DS), f32)] * 4
        + [pltpu.VMEM((KB, D), f32)] * 4
        + [pltpu.VMEM_SHARED((N_PAD, DS), f32)]
        + [pltpu.SemaphoreType.DMA] * 12
    ),
)(_sc_body)


def kernel(x, edge_index, W0, b0, ln_w, ln_b, W1, b1):
    src = edge_index[0].astype(i32)
    dst = edge_index[1].astype(i32)
    bad = src == dst
    srcp = jnp.where(bad, N, src)
    dstp = jnp.where(bad, N, dst)
    loop_idx = jnp.arange(N, dtype=i32)
    padv = jnp.full((E_PAD - E_TOT,), N, dtype=i32)
    src_full = jnp.concatenate([srcp, loop_idx, padv]).reshape(NW, NBATCH, KB)
    dst_full = jnp.concatenate([dstp, loop_idx, padv]).reshape(NW, NBATCH, KB)
    idx_full = jnp.stack([src_full, dst_full], axis=2)
    idx_full = jnp.pad(idx_full, ((0, 0), (0, 4), (0, 0), (0, 0)),
                       constant_values=N)

    x_pad = jnp.pad(x.astype(f32), ((0, N_PAD - N), (0, 0)))
    zz = jnp.zeros((ROWS_PER_TILE, DS), f32)

    xhat1, bt1 = _tc_pre(x_pad, W0)
    acc1 = _sc_attn(xhat1, bt1, idx_full, zz)
    xhat2, bt2 = _tc_mid(acc1, b0, ln_w, ln_b, W1)
    acc2 = _sc_attn(xhat2, bt2, idx_full, zz)
    out = _tc_final(acc2, b1)
    return out[:N]


# speedup vs baseline: 1.0004x; 1.0004x over previous
import functools

import jax
import jax.numpy as jnp
from jax import lax
from jax.experimental import pallas as pl
from jax.experimental.pallas import tpu as pltpu, tpu_sc as plsc

N = 10000
D = 128
E = 320000
THRESHOLD = 0.1

DS = D + 16
N_PAD = 10240
BLK = 512
GRID = N_PAD // BLK

NC = 2
NS = 16
NW = NC * NS
KB = 32
NBATCH = 324
EPW = KB * NBATCH
E_TOT = E + N
E_PAD = NW * EPW
ROWS_PER_TILE = N_PAD // NS

f32 = jnp.float32
i32 = jnp.int32


def _aux_cols(rn, rows):
    return jnp.concatenate(
        [rn, jnp.ones((rows, 1), f32), jnp.zeros((rows, 14), f32)], axis=1)


def _pre_body(x_ref, w_ref, xhat_ref, b_ref):
    x = x_ref[...]
    s = jnp.sum(x * x, axis=-1, keepdims=True)
    ni = jnp.maximum(jnp.sqrt(s), 1e-8)
    xhat_ref[...] = x / ni
    h = jnp.dot(x, w_ref[...], preferred_element_type=f32)
    rn = 1.0 / jnp.maximum(jnp.sqrt(jnp.sum(h * h, axis=-1, keepdims=True)), 1e-8)
    b_ref[:, 0:D] = h
    b_ref[:, D:DS] = _aux_cols(rn, BLK)


def _tc_pre(x_pad, W):
    return pl.pallas_call(
        _pre_body,
        grid=(GRID,),
        in_specs=[
            pl.BlockSpec((BLK, D), lambda i: (i, 0)),
            pl.BlockSpec((D, D), lambda i: (0, 0)),
        ],
        out_specs=[
            pl.BlockSpec((BLK, D), lambda i: (i, 0)),
            pl.BlockSpec((BLK, DS), lambda i: (i, 0)),
        ],
        out_shape=[
            jax.ShapeDtypeStruct((N_PAD, D), f32),
            jax.ShapeDtypeStruct((N_PAD, DS), f32),
        ],
    )(x_pad, W)


def _combine(acc_ref, b_ref):
    nm = acc_ref[0, :, 0:D] + acc_ref[1, :, 0:D]
    dd = acc_ref[0, :, D + 1:D + 2] + acc_ref[1, :, D + 1:D + 2] + 1e-16
    return nm / dd + b_ref[...]


def _mid_body(acc_ref, b_ref, lnw_ref, lnb_ref, w1_ref, xhat_ref, b2_ref):
    out = _combine(acc_ref, b_ref)
    mu = jnp.mean(out, axis=-1, keepdims=True)
    var = jnp.mean((out - mu) ** 2, axis=-1, keepdims=True)
    h1 = (out - mu) / jnp.sqrt(var + 1e-5) * lnw_ref[...] + lnb_ref[...]
    h1 = jnp.maximum(h1, 0.0)
    s = jnp.sum(h1 * h1, axis=-1, keepdims=True)
    ni = jnp.maximum(jnp.sqrt(s), 1e-8)
    xhat_ref[...] = h1 / ni
    h2 = jnp.dot(h1, w1_ref[...], preferred_element_type=f32)
    rn = 1.0 / jnp.maximum(jnp.sqrt(jnp.sum(h2 * h2, axis=-1, keepdims=True)), 1e-8)
    b2_ref[:, 0:D] = h2
    b2_ref[:, D:DS] = _aux_cols(rn, BLK)


def _tc_mid(acc, b0, ln_w, ln_b, W1):
    return pl.pallas_call(
        _mid_body,
        grid=(GRID,),
        in_specs=[
            pl.BlockSpec((NC, BLK, DS), lambda i: (0, i, 0)),
            pl.BlockSpec((1, D), lambda i: (0, 0)),
            pl.BlockSpec((1, D), lambda i: (0, 0)),
            pl.BlockSpec((1, D), lambda i: (0, 0)),
            pl.BlockSpec((D, D), lambda i: (0, 0)),
        ],
        out_specs=[
            pl.BlockSpec((BLK, D), lambda i: (i, 0)),
            pl.BlockSpec((BLK, DS), lambda i: (i, 0)),
        ],
        out_shape=[
            jax.ShapeDtypeStruct((N_PAD, D), f32),
            jax.ShapeDtypeStruct((N_PAD, DS), f32),
        ],
    )(acc, b0.reshape(1, D), ln_w.reshape(1, D), ln_b.reshape(1, D), W1)


def _final_body(acc_ref, b_ref, out_ref):
    out = _combine(acc_ref, b_ref)
    m = jnp.max(out, axis=-1, keepdims=True)
    z = out - m
    out_ref[...] = z - jnp.log(jnp.sum(jnp.exp(z), axis=-1, keepdims=True))


def _tc_final(acc, b1):
    return pl.pallas_call(
        _final_body,
        grid=(GRID,),
        in_specs=[
            pl.BlockSpec((NC, BLK, DS), lambda i: (0, i, 0)),
            pl.BlockSpec((1, D), lambda i: (0, 0)),
        ],
        out_specs=pl.BlockSpec((BLK, D), lambda i: (i, 0)),
        out_shape=jax.ShapeDtypeStruct((N_PAD, D), f32),
    )(acc, b1.reshape(1, D))


def _sc_body(xhat_hbm, srct_hbm, idx_hbm, zz_hbm,
             acc_hbm,
             idxb, bs0, bs1, bs2, bs3, xd0, xd1, xd2, xd3, spacc,
             gsem0, gsem1, gsem2, gsem3, ssem0, ssem1, ssem2, ssem3,
             isem0, isem1, isem2, isem3):
    c = lax.axis_index("c")
    s = lax.axis_index("s")
    wid = c * NS + s
    rbase = s * ROWS_PER_TILE

    bs = (bs0, bs1, bs2, bs3)
    xd = (xd0, xd1, xd2, xd3)
    gsem = (gsem0, gsem1, gsem2, gsem3)
    ssem = (ssem0, ssem1, ssem2, ssem3)
    isem = (isem0, isem1, isem2, isem3)

    pltpu.sync_copy(zz_hbm, spacc.at[pl.ds(rbase, ROWS_PER_TILE)])

    plsc.subcore_barrier()

    def compute(p):
        def edge_body(e, carry2):
            ch = [bs[p][e, pl.ds(cc * 16, 16)] for cc in range(8)]
            aux = bs[p][e, pl.ds(D, 16)]
            acc = ch[0] * xd[p][e, pl.ds(0, 16)]
            for cc in range(1, 8):
                acc = acc + ch[cc] * xd[p][e, pl.ds(cc * 16, 16)]
            w = jnp.sum(acc) * aux[0]
            w = jnp.where(w < THRESHOLD, f32(1e-6), w)
            for cc in range(8):
                bs[p][e, pl.ds(cc * 16, 16)] = ch[cc] * w
            bs[p][e, pl.ds(D, 16)] = aux * w
            return carry2

        lax.fori_loop(0, KB, edge_body, 0, unroll=2)

    def issue_gathers(b, p):
        sl = lax.rem(b, 8)
        pltpu.async_copy(srct_hbm.at[idxb.at[sl, 0]], bs[p], gsem[p])
        pltpu.async_copy(xhat_hbm.at[idxb.at[sl, 1]], xd[p], gsem[p])

    def drain_gathers(b, p):
        sl = lax.rem(b, 8)
        pltpu.make_async_copy(srct_hbm.at[idxb.at[sl, 0]], bs[p], gsem[p]).wait()
        pltpu.make_async_copy(xhat_hbm.at[idxb.at[sl, 1]], xd[p], gsem[p]).wait()

    def issue_scatter(b, p):
        sl = lax.rem(b, 8)
        pltpu.async_copy(bs[p], spacc.at[idxb.at[sl, 1]], ssem[p], add=True)

    def drain_scatter(b, p):
        sl = lax.rem(b, 8)
        pltpu.make_async_copy(bs[p], spacc.at[idxb.at[sl, 1]], ssem[p]).wait()

    def issue_idx(b, parity):
        sl = lax.rem(b, 8)
        pltpu.async_copy(idx_hbm.at[wid, b], idxb.at[sl], isem[parity])

    def drain_idx(b, parity):
        sl = lax.rem(b, 8)
        pltpu.make_async_copy(idx_hbm.at[wid, b], idxb.at[sl], isem[parity]).wait()

    pltpu.sync_copy(idx_hbm.at[wid, 0], idxb.at[0])
    pltpu.sync_copy(idx_hbm.at[wid, 1], idxb.at[1])
    issue_idx(2, 2)
    issue_idx(3, 3)
    issue_gathers(0, 0)
    issue_gathers(1, 1)

    def k_body(k, carry):
        for p in (0, 1, 2, 3):
            b = 4 * k + p

            def _prefetch_gathers():
                drain_idx(b + 2, (p + 2) % 4)
                issue_gathers(b + 2, (p + 2) % 4)

            drain_gathers(b, p)
            pl.when(b >= 2)(lambda: drain_scatter(b - 2, (p + 2) % 4))
            pl.when(b + 2 < NBATCH)(_prefetch_gathers)
            issue_idx(b + 4, p)
            compute(p)
            issue_scatter(b, p)
        return carry

    lax.fori_loop(0, NBATCH // 4, k_body, 0)

    drain_scatter(NBATCH - 2, 2)
    drain_scatter(NBATCH - 1, 3)
    for q in (0, 1, 2, 3):
        drain_idx(NBATCH + q, q)

    plsc.subcore_barrier()

    pltpu.sync_copy(spacc.at[pl.ds(rbase, ROWS_PER_TILE)],
                    acc_hbm.at[c, pl.ds(rbase, ROWS_PER_TILE)])


_sc_attn = functools.partial(
    pl.kernel,
    out_type=jax.ShapeDtypeStruct((NC, N_PAD, DS), f32),
    mesh=plsc.VectorSubcoreMesh(core_axis_name="c", subcore_axis_name="s"),
    compiler_params=pltpu.CompilerParams(needs_layout_passes=False,
                                         use_tc_tiling_on_sc=False),
    scratch_types=(
        [pltpu.VMEM((8, 2, KB), i32)]
        + [pltpu.VMEM((KB, DS), f32)] * 4
        + [pltpu.VMEM((KB, D), f32)] * 4
        + [pltpu.VMEM_SHARED((N_PAD, DS), f32)]
        + [pltpu.SemaphoreType.DMA] * 12
    ),
)(_sc_body)


def kernel(x, edge_index, W0, b0, ln_w, ln_b, W1, b1):
    src = edge_index[0].astype(i32)
    dst = edge_index[1].astype(i32)
    bad = src == dst
    srcp = jnp.where(bad, N, src)
    dstp = jnp.where(bad, N, dst)
    loop_idx = jnp.arange(N, dtype=i32)
    padv = jnp.full((E_PAD - E_TOT,), N, dtype=i32)
    src_full = jnp.concatenate([srcp, loop_idx, padv]).reshape(NW, NBATCH, KB)
    dst_full = jnp.concatenate([dstp, loop_idx, padv]).reshape(NW, NBATCH, KB)
    idx_full = jnp.stack([src_full, dst_full], axis=2)
    idx_full = jnp.pad(idx_full, ((0, 0), (0, 4), (0, 0), (0, 0)),
                       constant_values=N)

    x_pad = jnp.pad(x.astype(f32), ((0, N_PAD - N), (0, 0)))
    zz = jnp.zeros((ROWS_PER_TILE, DS), f32)

    xhat1, bt1 = _tc_pre(x_pad, W0)
    acc1 = _sc_attn(xhat1, bt1, idx_full, zz)
    xhat2, bt2 = _tc_mid(acc1, b0, ln_w, ln_b, W1)
    acc2 = _sc_attn(xhat2, bt2, idx_full, zz)
    out = _tc_final(acc2, b1)
    return out[:N]


# speedup vs baseline: 1.0037x; 1.0034x over previous
import functools

import jax
import jax.numpy as jnp
from jax import lax
from jax.experimental import pallas as pl
from jax.experimental.pallas import tpu as pltpu, tpu_sc as plsc

N = 10000
D = 128
E = 320000
THRESHOLD = 0.1

DS = D + 16
N_PAD = 10240
BLK = 512
GRID = N_PAD // BLK

NC = 2
NS = 16
NW = NC * NS
KB = 32
NBATCH = 324
EPW = KB * NBATCH
E_TOT = E + N
E_PAD = NW * EPW
ROWS_PER_TILE = N_PAD // NS

f32 = jnp.float32
i32 = jnp.int32


def _aux_cols(rn, rows):
    return jnp.concatenate(
        [rn, jnp.ones((rows, 1), f32), jnp.zeros((rows, 14), f32)], axis=1)


def _pre_body(x_ref, w_ref, xhat_ref, b_ref):
    x = x_ref[...]
    s = jnp.sum(x * x, axis=-1, keepdims=True)
    ni = jnp.maximum(jnp.sqrt(s), 1e-8)
    xhat_ref[...] = x / ni
    h = jnp.dot(x, w_ref[...], preferred_element_type=f32)
    rn = 1.0 / jnp.maximum(jnp.sqrt(jnp.sum(h * h, axis=-1, keepdims=True)), 1e-8)
    b_ref[:, 0:D] = h
    b_ref[:, D:DS] = _aux_cols(rn, BLK)


def _tc_pre(x_pad, W):
    return pl.pallas_call(
        _pre_body,
        grid=(GRID,),
        in_specs=[
            pl.BlockSpec((BLK, D), lambda i: (i, 0)),
            pl.BlockSpec((D, D), lambda i: (0, 0)),
        ],
        out_specs=[
            pl.BlockSpec((BLK, D), lambda i: (i, 0)),
            pl.BlockSpec((BLK, DS), lambda i: (i, 0)),
        ],
        out_shape=[
            jax.ShapeDtypeStruct((N_PAD, D), f32),
            jax.ShapeDtypeStruct((N_PAD, DS), f32),
        ],
    )(x_pad, W)


def _combine(acc_ref, b_ref):
    nm = acc_ref[0, :, 0:D] + acc_ref[1, :, 0:D]
    dd = acc_ref[0, :, D + 1:D + 2] + acc_ref[1, :, D + 1:D + 2] + 1e-16
    return nm / dd + b_ref[...]


def _mid_body(acc_ref, b_ref, lnw_ref, lnb_ref, w1_ref, xhat_ref, b2_ref):
    out = _combine(acc_ref, b_ref)
    mu = jnp.mean(out, axis=-1, keepdims=True)
    var = jnp.mean((out - mu) ** 2, axis=-1, keepdims=True)
    h1 = (out - mu) / jnp.sqrt(var + 1e-5) * lnw_ref[...] + lnb_ref[...]
    h1 = jnp.maximum(h1, 0.0)
    s = jnp.sum(h1 * h1, axis=-1, keepdims=True)
    ni = jnp.maximum(jnp.sqrt(s), 1e-8)
    xhat_ref[...] = h1 / ni
    h2 = jnp.dot(h1, w1_ref[...], preferred_element_type=f32)
    rn = 1.0 / jnp.maximum(jnp.sqrt(jnp.sum(h2 * h2, axis=-1, keepdims=True)), 1e-8)
    b2_ref[:, 0:D] = h2
    b2_ref[:, D:DS] = _aux_cols(rn, BLK)


def _tc_mid(acc, b0, ln_w, ln_b, W1):
    return pl.pallas_call(
        _mid_body,
        grid=(GRID,),
        in_specs=[
            pl.BlockSpec((NC, BLK, DS), lambda i: (0, i, 0)),
            pl.BlockSpec((1, D), lambda i: (0, 0)),
            pl.BlockSpec((1, D), lambda i: (0, 0)),
            pl.BlockSpec((1, D), lambda i: (0, 0)),
            pl.BlockSpec((D, D), lambda i: (0, 0)),
        ],
        out_specs=[
            pl.BlockSpec((BLK, D), lambda i: (i, 0)),
            pl.BlockSpec((BLK, DS), lambda i: (i, 0)),
        ],
        out_shape=[
            jax.ShapeDtypeStruct((N_PAD, D), f32),
            jax.ShapeDtypeStruct((N_PAD, DS), f32),
        ],
    )(acc, b0.reshape(1, D), ln_w.reshape(1, D), ln_b.reshape(1, D), W1)


def _final_body(acc_ref, b_ref, out_ref):
    out = _combine(acc_ref, b_ref)
    m = jnp.max(out, axis=-1, keepdims=True)
    z = out - m
    out_ref[...] = z - jnp.log(jnp.sum(jnp.exp(z), axis=-1, keepdims=True))


def _tc_final(acc, b1):
    return pl.pallas_call(
        _final_body,
        grid=(GRID,),
        in_specs=[
            pl.BlockSpec((NC, BLK, DS), lambda i: (0, i, 0)),
            pl.BlockSpec((1, D), lambda i: (0, 0)),
        ],
        out_specs=pl.BlockSpec((BLK, D), lambda i: (i, 0)),
        out_shape=jax.ShapeDtypeStruct((N_PAD, D), f32),
    )(acc, b1.reshape(1, D))


def _sc_body(xhat_hbm, srct_hbm, idx_hbm, zz_hbm,
             acc_hbm,
             idxb, bs0, bs1, bs2, bs3, xd0, xd1, xd2, xd3, spacc,
             gsem0, gsem1, gsem2, gsem3, ssem0, ssem1, ssem2, ssem3,
             isem0, isem1, isem2, isem3):
    c = lax.axis_index("c")
    s = lax.axis_index("s")
    wid = c * NS + s
    rbase = s * ROWS_PER_TILE

    bs = (bs0, bs1, bs2, bs3)
    xd = (xd0, xd1, xd2, xd3)
    gsem = (gsem0, gsem1, gsem2, gsem3)
    ssem = (ssem0, ssem1, ssem2, ssem3)
    isem = (isem0, isem1, isem2, isem3)

    pltpu.sync_copy(zz_hbm, spacc.at[pl.ds(rbase, ROWS_PER_TILE)])

    plsc.subcore_barrier()

    def compute(p):
        def edge_body(e, carry2):
            ch = [bs[p][e, pl.ds(cc * 16, 16)] for cc in range(8)]
            aux = bs[p][e, pl.ds(D, 16)]
            acc = ch[0] * xd[p][e, pl.ds(0, 16)]
            for cc in range(1, 8):
                acc = acc + ch[cc] * xd[p][e, pl.ds(cc * 16, 16)]
            w = jnp.sum(acc) * aux[0]
            w = jnp.where(w < THRESHOLD, f32(1e-6), w)
            for cc in range(8):
                bs[p][e, pl.ds(cc * 16, 16)] = ch[cc] * w
            bs[p][e, pl.ds(D, 16)] = aux * w
            return carry2

        lax.fori_loop(0, KB, edge_body, 0, unroll=2)

    def issue_gathers(b, p):
        sl = lax.rem(b, 8)
        pltpu.async_copy(srct_hbm.at[idxb.at[sl, 0]], bs[p], gsem[p])
        pltpu.async_copy(xhat_hbm.at[idxb.at[sl, 1]], xd[p], gsem[p])

    def drain_gathers(b, p):
        sl = lax.rem(b, 8)
        pltpu.make_async_copy(srct_hbm.at[idxb.at[sl, 0]], bs[p], gsem[p]).wait()
        pltpu.make_async_copy(xhat_hbm.at[idxb.at[sl, 1]], xd[p], gsem[p]).wait()

    def issue_scatter(b, p):
        sl = lax.rem(b, 8)
        pltpu.async_copy(bs[p], spacc.at[idxb.at[sl, 1]], ssem[p], add=True)

    def drain_scatter(b, p):
        sl = lax.rem(b, 8)
        pltpu.make_async_copy(bs[p], spacc.at[idxb.at[sl, 1]], ssem[p]).wait()

    def issue_idx(b, parity):
        sl = lax.rem(b, 8)
        pltpu.async_copy(idx_hbm.at[wid, b], idxb.at[sl], isem[parity])

    def drain_idx(b, parity):
        sl = lax.rem(b, 8)
        pltpu.make_async_copy(idx_hbm.at[wid, b], idxb.at[sl], isem[parity]).wait()

    pltpu.sync_copy(idx_hbm.at[wid, 0], idxb.at[0])
    pltpu.sync_copy(idx_hbm.at[wid, 1], idxb.at[1])
    issue_idx(2, 2)
    issue_idx(3, 3)
    issue_gathers(0, 0)
    issue_gathers(1, 1)

    def k_body(k, carry):
        for p in (0, 1, 2, 3):
            b = 4 * k + p

            def _prefetch_gathers():
                drain_idx(b + 2, (p + 2) % 4)
                issue_gathers(b + 2, (p + 2) % 4)

            drain_gathers(b, p)
            pl.when(b >= 2)(lambda: drain_scatter(b - 2, (p + 2) % 4))
            pl.when(b + 2 < NBATCH)(_prefetch_gathers)
            pl.when(b + 4 < NBATCH)(lambda: issue_idx(b + 4, p))
            compute(p)
            issue_scatter(b, p)
        return carry

    lax.fori_loop(0, NBATCH // 4, k_body, 0)

    drain_scatter(NBATCH - 2, 2)
    drain_scatter(NBATCH - 1, 3)

    plsc.subcore_barrier()

    pltpu.sync_copy(spacc.at[pl.ds(rbase, ROWS_PER_TILE)],
                    acc_hbm.at[c, pl.ds(rbase, ROWS_PER_TILE)])


_sc_attn = functools.partial(
    pl.kernel,
    out_type=jax.ShapeDtypeStruct((NC, N_PAD, DS), f32),
    mesh=plsc.VectorSubcoreMesh(core_axis_name="c", subcore_axis_name="s"),
    compiler_params=pltpu.CompilerParams(needs_layout_passes=False,
                                         use_tc_tiling_on_sc=False),
    scratch_types=(
        [pltpu.VMEM((8, 2, KB), i32)]
        + [pltpu.VMEM((KB, DS), f32)] * 4
        + [pltpu.VMEM((KB, D), f32)] * 4
        + [pltpu.VMEM_SHARED((N_PAD, DS), f32)]
        + [pltpu.SemaphoreType.DMA] * 12
    ),
)(_sc_body)


def kernel(x, edge_index, W0, b0, ln_w, ln_b, W1, b1):
    src = edge_index[0].astype(i32)
    dst = edge_index[1].astype(i32)
    bad = src == dst
    srcp = jnp.where(bad, N, src)
    dstp = jnp.where(bad, N, dst)
    loop_idx = jnp.arange(N, dtype=i32)
    padv = jnp.full((E_PAD - E_TOT,), N, dtype=i32)
    src_full = jnp.concatenate([srcp, loop_idx, padv]).reshape(NW, NBATCH, KB)
    dst_full = jnp.concatenate([dstp, loop_idx, padv]).reshape(NW, NBATCH, KB)
    idx_full = jnp.stack([src_full, dst_full], axis=2)

    x_pad = jnp.pad(x.astype(f32), ((0, N_PAD - N), (0, 0)))
    zz = jnp.zeros((ROWS_PER_TILE, DS), f32)

    xhat1, bt1 = _tc_pre(x_pad, W0)
    acc1 = _sc_attn(xhat1, bt1, idx_full, zz)
    xhat2, bt2 = _tc_mid(acc1, b0, ln_w, ln_b, W1)
    acc2 = _sc_attn(xhat2, bt2, idx_full, zz)
    out = _tc_final(acc2, b1)
    return out[:N]


# speedup vs baseline: 1.0358x; 1.0320x over previous
import functools

import jax
import jax.numpy as jnp
from jax import lax
from jax.experimental import pallas as pl
from jax.experimental.pallas import tpu as pltpu, tpu_sc as plsc

N = 10000
D = 128
E = 320000
THRESHOLD = 0.1

DS = D + 16
N_PAD = 10112
BLK = 632
GRID = N_PAD // BLK

NC = 2
NS = 16
NW = NC * NS
KB = 36
NBATCH = 288
EPW = KB * NBATCH
E_TOT = E + N
E_PAD = NW * EPW
ROWS_PER_TILE = N_PAD // NS

f32 = jnp.float32
i32 = jnp.int32


def _aux_cols(rn, rows):
    return jnp.concatenate(
        [rn, jnp.ones((rows, 1), f32), jnp.zeros((rows, 14), f32)], axis=1)


def _pre_body(x_ref, w_ref, xhat_ref, b_ref):
    x = x_ref[...]
    s = jnp.sum(x * x, axis=-1, keepdims=True)
    ni = jnp.maximum(jnp.sqrt(s), 1e-8)
    xhat_ref[...] = x / ni
    h = jnp.dot(x, w_ref[...], preferred_element_type=f32)
    rn = 1.0 / jnp.maximum(jnp.sqrt(jnp.sum(h * h, axis=-1, keepdims=True)), 1e-8)
    b_ref[:, 0:D] = h
    b_ref[:, D:DS] = _aux_cols(rn, BLK)


def _tc_pre(x_pad, W):
    return pl.pallas_call(
        _pre_body,
        grid=(GRID,),
        in_specs=[
            pl.BlockSpec((BLK, D), lambda i: (i, 0)),
            pl.BlockSpec((D, D), lambda i: (0, 0)),
        ],
        out_specs=[
            pl.BlockSpec((BLK, D), lambda i: (i, 0)),
            pl.BlockSpec((BLK, DS), lambda i: (i, 0)),
        ],
        out_shape=[
            jax.ShapeDtypeStruct((N_PAD, D), f32),
            jax.ShapeDtypeStruct((N_PAD, DS), f32),
        ],
    )(x_pad, W)


def _combine(acc_ref, b_ref):
    nm = acc_ref[0, :, 0:D] + acc_ref[1, :, 0:D]
    dd = acc_ref[0, :, D + 1:D + 2] + acc_ref[1, :, D + 1:D + 2] + 1e-16
    return nm / dd + b_ref[...]


def _mid_body(acc_ref, b_ref, lnw_ref, lnb_ref, w1_ref, xhat_ref, b2_ref):
    out = _combine(acc_ref, b_ref)
    mu = jnp.mean(out, axis=-1, keepdims=True)
    var = jnp.mean((out - mu) ** 2, axis=-1, keepdims=True)
    h1 = (out - mu) / jnp.sqrt(var + 1e-5) * lnw_ref[...] + lnb_ref[...]
    h1 = jnp.maximum(h1, 0.0)
    s = jnp.sum(h1 * h1, axis=-1, keepdims=True)
    ni = jnp.maximum(jnp.sqrt(s), 1e-8)
    xhat_ref[...] = h1 / ni
    h2 = jnp.dot(h1, w1_ref[...], preferred_element_type=f32)
    rn = 1.0 / jnp.maximum(jnp.sqrt(jnp.sum(h2 * h2, axis=-1, keepdims=True)), 1e-8)
    b2_ref[:, 0:D] = h2
    b2_ref[:, D:DS] = _aux_cols(rn, BLK)


def _tc_mid(acc, b0, ln_w, ln_b, W1):
    return pl.pallas_call(
        _mid_body,
        grid=(GRID,),
        in_specs=[
            pl.BlockSpec((NC, BLK, DS), lambda i: (0, i, 0)),
            pl.BlockSpec((1, D), lambda i: (0, 0)),
            pl.BlockSpec((1, D), lambda i: (0, 0)),
            pl.BlockSpec((1, D), lambda i: (0, 0)),
            pl.BlockSpec((D, D), lambda i: (0, 0)),
        ],
        out_specs=[
            pl.BlockSpec((BLK, D), lambda i: (i, 0)),
            pl.BlockSpec((BLK, DS), lambda i: (i, 0)),
        ],
        out_shape=[
            jax.ShapeDtypeStruct((N_PAD, D), f32),
            jax.ShapeDtypeStruct((N_PAD, DS), f32),
        ],
    )(acc, b0.reshape(1, D), ln_w.reshape(1, D), ln_b.reshape(1, D), W1)


def _final_body(acc_ref, b_ref, out_ref):
    out = _combine(acc_ref, b_ref)
    m = jnp.max(out, axis=-1, keepdims=True)
    z = out - m
    out_ref[...] = z - jnp.log(jnp.sum(jnp.exp(z), axis=-1, keepdims=True))


def _tc_final(acc, b1):
    return pl.pallas_call(
        _final_body,
        grid=(GRID,),
        in_specs=[
            pl.BlockSpec((NC, BLK, DS), lambda i: (0, i, 0)),
            pl.BlockSpec((1, D), lambda i: (0, 0)),
        ],
        out_specs=pl.BlockSpec((BLK, D), lambda i: (i, 0)),
        out_shape=jax.ShapeDtypeStruct((N_PAD, D), f32),
    )(acc, b1.reshape(1, D))


def _sc_body(xhat_hbm, srct_hbm, idx_hbm, zz_hbm,
             acc_hbm,
             idxb, bs0, bs1, bs2, bs3, xd0, xd1, xd2, xd3, spacc,
             gsem0, gsem1, gsem2, gsem3, ssem0, ssem1, ssem2, ssem3,
             isem0, isem1, isem2, isem3):
    c = lax.axis_index("c")
    s = lax.axis_index("s")
    wid = c * NS + s
    rbase = s * ROWS_PER_TILE

    bs = (bs0, bs1, bs2, bs3)
    xd = (xd0, xd1, xd2, xd3)
    gsem = (gsem0, gsem1, gsem2, gsem3)
    ssem = (ssem0, ssem1, ssem2, ssem3)
    isem = (isem0, isem1, isem2, isem3)

    pltpu.sync_copy(zz_hbm, spacc.at[pl.ds(rbase, ROWS_PER_TILE)])

    plsc.subcore_barrier()

    def compute(p):
        def edge_body(e, carry2):
            ch = [bs[p][e, pl.ds(cc * 16, 16)] for cc in range(8)]
            aux = bs[p][e, pl.ds(D, 16)]
            acc = ch[0] * xd[p][e, pl.ds(0, 16)]
            for cc in range(1, 8):
                acc = acc + ch[cc] * xd[p][e, pl.ds(cc * 16, 16)]
            w = jnp.sum(acc) * aux[0]
            w = jnp.where(w < THRESHOLD, f32(1e-6), w)
            for cc in range(8):
                bs[p][e, pl.ds(cc * 16, 16)] = ch[cc] * w
            bs[p][e, pl.ds(D, 16)] = aux * w
            return carry2

        lax.fori_loop(0, KB, edge_body, 0, unroll=2)

    def issue_gathers(b, p):
        sl = lax.rem(b, 8)
        pltpu.async_copy(srct_hbm.at[idxb.at[sl, 0]], bs[p], gsem[p])
        pltpu.async_copy(xhat_hbm.at[idxb.at[sl, 1]], xd[p], gsem[p])

    def drain_gathers(b, p):
        sl = lax.rem(b, 8)
        pltpu.make_async_copy(srct_hbm.at[idxb.at[sl, 0]], bs[p], gsem[p]).wait()
        pltpu.make_async_copy(xhat_hbm.at[idxb.at[sl, 1]], xd[p], gsem[p]).wait()

    def issue_scatter(b, p):
        sl = lax.rem(b, 8)
        pltpu.async_copy(bs[p], spacc.at[idxb.at[sl, 1]], ssem[p], add=True)

    def drain_scatter(b, p):
        sl = lax.rem(b, 8)
        pltpu.make_async_copy(bs[p], spacc.at[idxb.at[sl, 1]], ssem[p]).wait()

    def issue_idx(b, parity):
        sl = lax.rem(b, 8)
        pltpu.async_copy(idx_hbm.at[wid, b], idxb.at[sl], isem[parity])

    def drain_idx(b, parity):
        sl = lax.rem(b, 8)
        pltpu.make_async_copy(idx_hbm.at[wid, b], idxb.at[sl], isem[parity]).wait()

    pltpu.sync_copy(idx_hbm.at[wid, 0], idxb.at[0])
    pltpu.sync_copy(idx_hbm.at[wid, 1], idxb.at[1])
    issue_idx(2, 2)
    issue_idx(3, 3)
    issue_gathers(0, 0)
    issue_gathers(1, 1)

    def k_body(k, carry):
        for p in (0, 1, 2, 3):
            b = 4 * k + p

            def _prefetch_gathers():
                drain_idx(b + 2, (p + 2) % 4)
                issue_gathers(b + 2, (p + 2) % 4)

            drain_gathers(b, p)
            pl.when(b >= 2)(lambda: drain_scatter(b - 2, (p + 2) % 4))
            pl.when(b + 2 < NBATCH)(_prefetch_gathers)
            pl.when(b + 4 < NBATCH)(lambda: issue_idx(b + 4, p))
            compute(p)
            issue_scatter(b, p)
        return carry

    lax.fori_loop(0, NBATCH // 4, k_body, 0)

    drain_scatter(NBATCH - 2, 2)
    drain_scatter(NBATCH - 1, 3)

    plsc.subcore_barrier()

    pltpu.sync_copy(spacc.at[pl.ds(rbase, ROWS_PER_TILE)],
                    acc_hbm.at[c, pl.ds(rbase, ROWS_PER_TILE)])


_sc_attn = functools.partial(
    pl.kernel,
    out_type=jax.ShapeDtypeStruct((NC, N_PAD, DS), f32),
    mesh=plsc.VectorSubcoreMesh(core_axis_name="c", subcore_axis_name="s"),
    compiler_params=pltpu.CompilerParams(needs_layout_passes=False,
                                         use_tc_tiling_on_sc=False),
    scratch_types=(
        [pltpu.VMEM((8, 2, KB), i32)]
        + [pltpu.VMEM((KB, DS), f32)] * 4
        + [pltpu.VMEM((KB, D), f32)] * 4
        + [pltpu.VMEM_SHARED((N_PAD, DS), f32)]
        + [pltpu.SemaphoreType.DMA] * 12
    ),
)(_sc_body)


def kernel(x, edge_index, W0, b0, ln_w, ln_b, W1, b1):
    src = edge_index[0].astype(i32)
    dst = edge_index[1].astype(i32)
    bad = src == dst
    srcp = jnp.where(bad, N, src)
    dstp = jnp.where(bad, N, dst)
    loop_idx = jnp.arange(N, dtype=i32)
    padv = jnp.full((E_PAD - E_TOT,), N, dtype=i32)
    src_full = jnp.concatenate([srcp, loop_idx, padv]).reshape(NW, NBATCH, KB)
    dst_full = jnp.concatenate([dstp, loop_idx, padv]).reshape(NW, NBATCH, KB)
    idx_full = jnp.stack([src_full, dst_full], axis=2)

    x_pad = jnp.pad(x.astype(f32), ((0, N_PAD - N), (0, 0)))
    zz = jnp.zeros((ROWS_PER_TILE, DS), f32)

    xhat1, bt1 = _tc_pre(x_pad, W0)
    acc1 = _sc_attn(xhat1, bt1, idx_full, zz)
    xhat2, bt2 = _tc_mid(acc1, b0, ln_w, ln_b, W1)
    acc2 = _sc_attn(xhat2, bt2, idx_full, zz)
    out = _tc_final(acc2, b1)
    return out[:N]
